# baseline (speedup 1.0000x reference)
import functools

import jax
import jax.numpy as jnp
from jax import lax
from jax.experimental import pallas as pl
from jax.experimental.pallas import tpu as pltpu
from jax.experimental.pallas import tpu_sc as plsc

N_STRUCT = 512
LANES = 16
K = 128


def _lower_bound(gather_fn, targets, n, steps):
    lo = jnp.zeros((LANES,), jnp.int32)
    hi = jnp.full((LANES,), n, jnp.int32)
    for _ in range(steps):
        active = lo < hi
        mid = jnp.right_shift(lo + hi, 1)
        midc = jnp.minimum(mid, n - 1)
        vals = gather_fn(midc)
        pred = vals < targets
        lo = jnp.where(active & pred, mid + 1, lo)
        hi = jnp.where(active & (~pred), mid, hi)
    return lo


def _steps_for(n):
    s = 1
    while (1 << s) < n:
        s += 1
    return s + 1


def _make_body(n_rows, n_sample, hidden):
    coarse_steps = _steps_for(n_sample)
    fine_steps = _steps_for(K)
    p_off = n_sample
    w2_off = p_off + hidden
    b2_off = w2_off + hidden * 6

    def f32_gather(ref, idx):
        return plsc.bitcast(plsc.load_gather(ref, [idx]), jnp.float32)

    def body(batch2d_hbm, aux_hbm, out_hbm,
             aux_v, idx_lo_v, idx_up_v, rows_lo_v, rows_up_v,
             counts_v, v_v, out_v, sem_lo, sem_up):
        wid = lax.axis_index("s") * 2 + lax.axis_index("c")
        pltpu.sync_copy(aux_hbm, aux_v)
        iota = lax.iota(jnp.int32, LANES)

        t_lo = wid * LANES + iota
        t_up = t_lo + 1

        def coarse(idx):
            return plsc.load_gather(aux_v, [idx])

        s_lo = _lower_bound(coarse, t_lo, n_sample, coarse_steps)
        s_up = _lower_bound(coarse, t_up, n_sample, coarse_steps)

        r_lo = jnp.clip(s_lo - 1, 0, n_rows - 1)
        r_up = jnp.clip(s_up - 1, 0, n_rows - 1)
        idx_lo_v[...] = r_lo
        idx_up_v[...] = r_up
        cp_lo = pltpu.async_copy(batch2d_hbm.at[idx_lo_v], rows_lo_v, sem_lo)
        cp_up = pltpu.async_copy(batch2d_hbm.at[idx_up_v], rows_up_v, sem_up)

        accs = [jnp.zeros((LANES,), jnp.float32) for _ in range(6)]
        for c in range(hidden // LANES):
            x = plsc.bitcast(aux_v[pl.ds(p_off + c * LANES, LANES)],
                             jnp.float32)
            s = x / (1.0 + jnp.exp(-x))
            row = w2_off + (c * LANES + iota) * 6
            for j in range(6):
                accs[j] = accs[j] + s * f32_gather(aux_v, row + j)
        b2g = f32_gather(aux_v, b2_off + jnp.minimum(iota, 5))
        v = jnp.where(iota < 6, b2g, 0.0)
        for j in range(6):
            v = jnp.where(iota == j, v + jnp.sum(accs[j]), v)
        v_v[...] = v

        cp_lo.wait()
        cp_up.wait()

        def fine(rows_v, targets, s_idx, r):
            def g(off):
                return plsc.load_gather(rows_v, [iota, off])
            off = _lower_bound(g, targets, K, fine_steps)
            return jnp.where(s_idx == 0, 0, r * K + off)

        pos_lo = fine(rows_lo_v, t_lo, s_lo, r_lo)
        pos_up = fine(rows_up_v, t_up, s_up, r_up)
        counts_v[...] = (pos_up - pos_lo).astype(jnp.float32)

        for k in range(6):
            p = k * LANES + iota
            b_local = p // 6
            j = p - 6 * b_local
            cnt = plsc.load_gather(counts_v, [b_local])
            vv = plsc.load_gather(v_v, [j])
            plsc.store_scatter(out_v, [b_local, j], cnt * vv)
        pltpu.sync_copy(out_v, out_hbm.at[pl.ds(wid * LANES, LANES), :])

    return body


def kernel(pos, batch, atomic_numbers, W1, b1, W2, b2):
    n = pos.shape[0]
    hidden = b1.shape[0]

    batch_i32 = batch.astype(jnp.int32)
    n_rows = -(-n // K)
    n_pad = n_rows * K
    batch_pad = jnp.concatenate(
        [batch_i32, jnp.full((n_pad - n,), N_STRUCT, jnp.int32)])
    batch2d = batch_pad.reshape(n_rows, K)
    n_sample = -(-(n_rows + 5) // 16) * 16

    p_len = hidden + hidden * 6 + 6
    p_pad = -(-p_len // 16) * 16
    params = jnp.concatenate(
        [b1.astype(jnp.float32),
         jnp.reshape(W2.astype(jnp.float32), (-1,)),
         b2.astype(jnp.float32),
         jnp.zeros((p_pad - p_len,), jnp.float32)])
    aux = jnp.concatenate(
        [batch_pad[::K],
         jnp.full((n_sample - n_rows,), N_STRUCT, jnp.int32),
         lax.bitcast_convert_type(params, jnp.int32)])

    mesh = plsc.VectorSubcoreMesh(core_axis_name="c", subcore_axis_name="s")
    run = functools.partial(
        pl.kernel,
        mesh=mesh,
        compiler_params=pltpu.CompilerParams(needs_layout_passes=False),
        out_type=jax.ShapeDtypeStruct((N_STRUCT, 6), jnp.float32),
        scratch_types=[
            pltpu.VMEM((n_sample + p_pad,), jnp.int32),
            pltpu.VMEM((LANES,), jnp.int32),
            pltpu.VMEM((LANES,), jnp.int32),
            pltpu.VMEM((LANES, K), jnp.int32),
            pltpu.VMEM((LANES, K), jnp.int32),
            pltpu.VMEM((LANES,), jnp.float32),
            pltpu.VMEM((LANES,), jnp.float32),
            pltpu.VMEM((LANES, 6), jnp.float32),
            pltpu.SemaphoreType.DMA,
            pltpu.SemaphoreType.DMA,
        ],
    )(_make_body(n_rows, n_sample, hidden))

    forces = jnp.zeros((n, 3), jnp.float32)
    energy = jnp.zeros((N_STRUCT,), jnp.float32)
    stress = run(batch2d, aux)
    return (forces, energy, stress)


# speedup vs baseline: 1.0076x; 1.0076x over previous
import functools

import jax
import jax.numpy as jnp
from jax import lax
from jax.experimental import pallas as pl
from jax.experimental.pallas import tpu as pltpu
from jax.experimental.pallas import tpu_sc as plsc

N_STRUCT = 512
LANES = 16
K = 128


def _lower_bound(gather_fn, targets, n, steps):
    lo = jnp.zeros((LANES,), jnp.int32)
    hi = jnp.full((LANES,), n, jnp.int32)
    for _ in range(steps):
        active = lo < hi
        mid = jnp.right_shift(lo + hi, 1)
        midc = jnp.minimum(mid, n - 1)
        vals = gather_fn(midc)
        pred = vals < targets
        lo = jnp.where(active & pred, mid + 1, lo)
        hi = jnp.where(active & (~pred), mid, hi)
    return lo


def _steps_for(n):
    s = 1
    while (1 << s) < n:
        s += 1
    return s + 1


def _make_body(n_rows, n_sample, hidden):
    coarse_steps = _steps_for(n_sample)
    fine_steps = _steps_for(K)
    p_off = n_sample
    w2_off = p_off + hidden
    b2_off = w2_off + hidden * 6

    def f32_gather(ref, idx):
        return plsc.bitcast(plsc.load_gather(ref, [idx]), jnp.float32)

    def body(batch2d_hbm, aux_hbm, out_hbm,
             aux_v, idx_lo_v, idx_up_v, rows_lo_v, rows_up_v,
             counts_v, v_v, out_v, sem_lo, sem_up):
        wid = lax.axis_index("s") * 2 + lax.axis_index("c")
        pltpu.sync_copy(aux_hbm, aux_v)
        iota = lax.iota(jnp.int32, LANES)

        t_lo = wid * LANES + iota
        t_up = t_lo + 1

        def coarse(idx):
            return plsc.load_gather(aux_v, [idx])

        s_lo = _lower_bound(coarse, t_lo, n_sample, coarse_steps)
        s_up = _lower_bound(coarse, t_up, n_sample, coarse_steps)

        r_lo = jnp.clip(s_lo - 1, 0, n_rows - 1)
        r_up = jnp.clip(s_up - 1, 0, n_rows - 1)
        idx_lo_v[...] = r_lo
        idx_up_v[...] = r_up
        cp_lo = pltpu.async_copy(batch2d_hbm.at[idx_lo_v], rows_lo_v, sem_lo)
        cp_up = pltpu.async_copy(batch2d_hbm.at[idx_up_v], rows_up_v, sem_up)

        accs = [jnp.zeros((LANES,), jnp.float32) for _ in range(6)]
        for c in range(hidden // LANES):
            x = plsc.bitcast(aux_v[pl.ds(p_off + c * LANES, LANES)],
                             jnp.float32)
            s = x / (1.0 + jnp.exp(-x))
            row = w2_off + (c * LANES + iota) * 6
            for j in range(6):
                accs[j] = accs[j] + s * f32_gather(aux_v, row + j)
        b2g = f32_gather(aux_v, b2_off + jnp.minimum(iota, 5))
        v = jnp.where(iota < 6, b2g, 0.0)
        for j in range(6):
            v = jnp.where(iota == j, v + jnp.sum(accs[j]), v)
        v_v[...] = v

        cp_lo.wait()
        cp_up.wait()

        def fine(rows_v, targets, s_idx, r):
            def g(off):
                return plsc.load_gather(rows_v, [iota, off])
            off = _lower_bound(g, targets, K, fine_steps)
            return jnp.where(s_idx == 0, 0, r * K + off)

        pos_lo = fine(rows_lo_v, t_lo, s_lo, r_lo)
        pos_up = fine(rows_up_v, t_up, s_up, r_up)
        counts_v[...] = (pos_up - pos_lo).astype(jnp.float32)

        for k in range(6):
            p = k * LANES + iota
            b_local = p // 6
            j = p - 6 * b_local
            cnt = plsc.load_gather(counts_v, [b_local])
            vv = plsc.load_gather(v_v, [j])
            out_v[pl.ds(k * LANES, LANES)] = cnt * vv
        pltpu.sync_copy(out_v, out_hbm.at[pl.ds(wid * LANES * 6, LANES * 6)])

    return body


def kernel(pos, batch, atomic_numbers, W1, b1, W2, b2):
    n = pos.shape[0]
    hidden = b1.shape[0]

    batch_i32 = batch.astype(jnp.int32)
    n_rows = -(-n // K)
    n_pad = n_rows * K
    batch_pad = jnp.concatenate(
        [batch_i32, jnp.full((n_pad - n,), N_STRUCT, jnp.int32)])
    batch2d = batch_pad.reshape(n_rows, K)
    n_sample = -(-(n_rows + 5) // 16) * 16

    p_len = hidden + hidden * 6 + 6
    p_pad = -(-p_len // 16) * 16
    params = jnp.concatenate(
        [b1.astype(jnp.float32),
         jnp.reshape(W2.astype(jnp.float32), (-1,)),
         b2.astype(jnp.float32),
         jnp.zeros((p_pad - p_len,), jnp.float32)])
    aux = jnp.concatenate(
        [batch_pad[::K],
         jnp.full((n_sample - n_rows,), N_STRUCT, jnp.int32),
         lax.bitcast_convert_type(params, jnp.int32)])

    mesh = plsc.VectorSubcoreMesh(core_axis_name="c", subcore_axis_name="s")
    run = functools.partial(
        pl.kernel,
        mesh=mesh,
        compiler_params=pltpu.CompilerParams(needs_layout_passes=False),
        out_type=jax.ShapeDtypeStruct((N_STRUCT * 6,), jnp.float32),
        scratch_types=[
            pltpu.VMEM((n_sample + p_pad,), jnp.int32),
            pltpu.VMEM((LANES,), jnp.int32),
            pltpu.VMEM((LANES,), jnp.int32),
            pltpu.VMEM((LANES, K), jnp.int32),
            pltpu.VMEM((LANES, K), jnp.int32),
            pltpu.VMEM((LANES,), jnp.float32),
            pltpu.VMEM((LANES,), jnp.float32),
            pltpu.VMEM((LANES * 6,), jnp.float32),
            pltpu.SemaphoreType.DMA,
            pltpu.SemaphoreType.DMA,
        ],
    )(_make_body(n_rows, n_sample, hidden))

    forces = jnp.zeros((n, 3), jnp.float32)
    energy = jnp.zeros((N_STRUCT,), jnp.float32)
    stress = run(batch2d, aux).reshape(N_STRUCT, 6)
    return (forces, energy, stress)


# speedup vs baseline: 1.0357x; 1.0278x over previous
import functools

import jax
import jax.numpy as jnp
from jax import lax
from jax.experimental import pallas as pl
from jax.experimental.pallas import tpu as pltpu
from jax.experimental.pallas import tpu_sc as plsc

N_STRUCT = 512
LANES = 16
K = 128


def _lower_bound(gather_fn, targets, n, steps):
    lo = jnp.zeros((LANES,), jnp.int32)
    hi = jnp.full((LANES,), n, jnp.int32)
    for _ in range(steps):
        active = lo < hi
        mid = jnp.right_shift(lo + hi, 1)
        midc = jnp.minimum(mid, n - 1)
        vals = gather_fn(midc)
        pred = vals < targets
        lo = jnp.where(active & pred, mid + 1, lo)
        hi = jnp.where(active & (~pred), mid, hi)
    return lo


def _steps_for(n):
    s = 1
    while (1 << s) < n:
        s += 1
    return s + 1


def _make_body(n, n_rows, n_sample, hidden):
    coarse_steps = _steps_for(n_sample)
    fine_steps = _steps_for(K)
    p_off = n_sample
    w2_off = p_off + hidden
    b2_off = w2_off + hidden * 6

    def f32_gather(ref, idx):
        return plsc.bitcast(plsc.load_gather(ref, [idx]), jnp.float32)

    def body(batch_hbm, aux_hbm, out_hbm,
             aux_v, rows_lo_v, rows_up_v,
             counts_v, v_v, out_v, sem_lo, sem_up):
        wid = lax.axis_index("s") * 2 + lax.axis_index("c")
        pltpu.sync_copy(aux_hbm, aux_v)
        iota = lax.iota(jnp.int32, LANES)

        t_lo = wid * LANES + iota
        t_up = t_lo + 1

        def coarse(idx):
            return plsc.load_gather(aux_v, [idx])

        s_lo = _lower_bound(coarse, t_lo, n_sample, coarse_steps)
        s_up = _lower_bound(coarse, t_up, n_sample, coarse_steps)

        r_lo = jnp.clip(s_lo - 1, 0, n_rows - 1)
        r_up = jnp.clip(s_up - 1, 0, n_rows - 1)
        w_lo = jnp.minimum(r_lo * K, n - K)
        w_up = jnp.minimum(r_up * K, n - K)
        cps = []
        for l in range(LANES):
            o_lo = pl.multiple_of(w_lo[l], 8)
            o_up = pl.multiple_of(w_up[l], 8)
            cps.append(pltpu.async_copy(
                batch_hbm.at[pl.ds(o_lo, K)], rows_lo_v.at[l], sem_lo))
            cps.append(pltpu.async_copy(
                batch_hbm.at[pl.ds(o_up, K)], rows_up_v.at[l], sem_up))

        accs = [jnp.zeros((LANES,), jnp.float32) for _ in range(6)]
        for c in range(hidden // LANES):
            x = plsc.bitcast(aux_v[pl.ds(p_off + c * LANES, LANES)],
                             jnp.float32)
            s = x / (1.0 + jnp.exp(-x))
            row = w2_off + (c * LANES + iota) * 6
            for j in range(6):
                accs[j] = accs[j] + s * f32_gather(aux_v, row + j)
        b2g = f32_gather(aux_v, b2_off + jnp.minimum(iota, 5))
        v = jnp.where(iota < 6, b2g, 0.0)
        for j in range(6):
            v = jnp.where(iota == j, v + jnp.sum(accs[j]), v)
        v_v[...] = v

        for cp in cps:
            cp.wait()

        def fine(rows_v, targets, s_idx, w):
            def g(off):
                return plsc.load_gather(rows_v, [iota, off])
            off = _lower_bound(g, targets, K, fine_steps)
            return jnp.where(s_idx == 0, 0, w + off)

        pos_lo = fine(rows_lo_v, t_lo, s_lo, w_lo)
        pos_up = fine(rows_up_v, t_up, s_up, w_up)
        counts_v[...] = (pos_up - pos_lo).astype(jnp.float32)

        for k in range(6):
            p = k * LANES + iota
            b_local = p // 6
            j = p - 6 * b_local
            cnt = plsc.load_gather(counts_v, [b_local])
            vv = plsc.load_gather(v_v, [j])
            out_v[pl.ds(k * LANES, LANES)] = cnt * vv
        pltpu.sync_copy(out_v, out_hbm.at[pl.ds(wid * LANES * 6, LANES * 6)])

    return body


def kernel(pos, batch, atomic_numbers, W1, b1, W2, b2):
    n = pos.shape[0]
    hidden = b1.shape[0]

    batch_i32 = batch.astype(jnp.int32)
    n_rows = -(-n // K)
    n_sample = -(-(n_rows + 5) // 16) * 16

    p_len = hidden + hidden * 6 + 6
    p_pad = -(-p_len // 16) * 16
    params = jnp.concatenate(
        [b1.astype(jnp.float32),
         jnp.reshape(W2.astype(jnp.float32), (-1,)),
         b2.astype(jnp.float32),
         jnp.zeros((p_pad - p_len,), jnp.float32)])
    aux = jnp.concatenate(
        [batch_i32[::K],
         jnp.full((n_sample - n_rows,), N_STRUCT, jnp.int32),
         lax.bitcast_convert_type(params, jnp.int32)])

    mesh = plsc.VectorSubcoreMesh(core_axis_name="c", subcore_axis_name="s")
    run = functools.partial(
        pl.kernel,
        mesh=mesh,
        compiler_params=pltpu.CompilerParams(needs_layout_passes=False),
        out_type=jax.ShapeDtypeStruct((N_STRUCT * 6,), jnp.float32),
        scratch_types=[
            pltpu.VMEM((n_sample + p_pad,), jnp.int32),
            pltpu.VMEM((LANES, K), jnp.int32),
            pltpu.VMEM((LANES, K), jnp.int32),
            pltpu.VMEM((LANES,), jnp.float32),
            pltpu.VMEM((LANES,), jnp.float32),
            pltpu.VMEM((LANES * 6,), jnp.float32),
            pltpu.SemaphoreType.DMA,
            pltpu.SemaphoreType.DMA,
        ],
    )(_make_body(n, n_rows, n_sample, hidden))

    forces = jnp.zeros((n, 3), jnp.float32)
    energy = jnp.zeros((N_STRUCT,), jnp.float32)
    stress = run(batch_i32, aux).reshape(N_STRUCT, 6)
    return (forces, energy, stress)
